# baseline (speedup 1.0000x reference)
import functools

import jax
import jax.numpy as jnp
from jax import lax
from jax.experimental import pallas as pl
from jax.experimental.pallas import tpu as pltpu
from jax.experimental.pallas import tpu_sc as plsc

B = 4096
L = 200
H = 64
VOC = 1000000
NC = 2
NS = 16
NW = NC * NS
BPW = B // NW
LANES = 16
NBUF = 4
LEG = 128

VBLK = 128
NBLK_MAIN = 244
V_MAIN = NBLK_MAIN * NW * VBLK

_PICK_DNUMS = lax.GatherDimensionNumbers(
    offset_dims=(), collapsed_slice_dims=(0,), start_index_map=(0,)
)


def _bcast_lane(vec, lane_idx):
    return lax.gather(
        vec,
        lane_idx[:, None],
        _PICK_DNUMS,
        (1,),
        mode=lax.GatherScatterMode.PROMISE_IN_BOUNDS,
    )


def _mesh():
    return plsc.VectorSubcoreMesh(
        core_axis_name="c", subcore_axis_name="s", num_cores=NC, num_subcores=NS
    )


NBLK = (VOC + 2 * H - 1) // (2 * H)
VOCP = NBLK * 2 * H


def _tc_repack_body(x_ref, o_ref):
    x = x_ref[...]
    o_ref[...] = jnp.concatenate([x[:, 0:H].T, x[:, H : 2 * H].T], axis=1)


_tc_repack = pl.pallas_call(
    _tc_repack_body,
    grid=(NBLK,),
    in_specs=[pl.BlockSpec((H, 2 * H), lambda i: (0, i))],
    out_specs=pl.BlockSpec((H, 2 * H), lambda i: (i, 0)),
    out_shape=jax.ShapeDtypeStruct((VOCP // 2, 2 * H), jnp.float32),
)


@functools.partial(
    pl.kernel,
    out_type=jax.ShapeDtypeStruct((B, L), jnp.float32),
    mesh=_mesh(),
    compiler_params=pltpu.CompilerParams(
        needs_layout_passes=False, use_tc_tiling_on_sc=False
    ),
    scratch_types=[
        pltpu.VMEM((BPW, H), jnp.float32),
        pltpu.VMEM((BPW, L), jnp.int32),
        pltpu.VMEM((NBUF, 208), jnp.int32),
        pltpu.VMEM((NBUF, L, H), jnp.float32),
        pltpu.VMEM((NBUF, 208), jnp.float32),
        pltpu.SemaphoreType.DMA,
        pltpu.SemaphoreType.DMA,
    ],
)
def _entity_head(
    hid_hbm, kb_hbm, tab_hbm, out_hbm, hid_v, idx_v, idxs_v, rows_v, outb_v,
    gsem, osem,
):
    wid = lax.axis_index("s") * NC + lax.axis_index("c")
    b0 = wid * BPW
    pltpu.sync_copy(hid_hbm.at[pl.ds(b0, BPW)], hid_v)
    pltpu.sync_copy(kb_hbm.at[pl.ds(b0, BPW)], idx_v)

    def fire_gather(bl, slot):
        for k in range(L // LANES + 1):
            t0 = min(k * LANES, L - LANES)
            v = idx_v[bl, pl.ds(t0, LANES)]
            u = jnp.bitwise_and(v, 127)
            f = v + u - jnp.where(u >= H, 127, 0)
            idxs_v[slot, pl.ds(t0, LANES)] = f
        pltpu.async_copy(
            tab_hbm.at[idxs_v.at[slot, pl.ds(0, LEG)]],
            rows_v.at[slot, pl.ds(0, LEG)],
            gsem,
        )
        pltpu.async_copy(
            tab_hbm.at[idxs_v.at[slot, pl.ds(LEG, L - LEG)]],
            rows_v.at[slot, pl.ds(LEG, L - LEG)],
            gsem,
        )

    for p in range(NBUF):
        fire_gather(p, p)

    def batch_body(bl, carry):
        slot = lax.rem(bl, NBUF)
        pltpu.make_async_copy(
            tab_hbm.at[idxs_v.at[slot, pl.ds(0, LEG)]],
            rows_v.at[slot, pl.ds(0, LEG)],
            gsem,
        ).wait()
        pltpu.make_async_copy(
            tab_hbm.at[idxs_v.at[slot, pl.ds(LEG, L - LEG)]],
            rows_v.at[slot, pl.ds(LEG, L - LEG)],
            gsem,
        ).wait()

        @pl.when(bl >= NBUF)
        def _():
            pltpu.make_async_copy(
                outb_v.at[slot, pl.ds(0, L)], out_hbm.at[b0 + bl - NBUF], osem
            ).wait()

        hv = [hid_v[bl, pl.ds(c * LANES, LANES)] for c in range(H // LANES)]
        lane_iota = lax.iota(jnp.int32, LANES)
        pick15 = jnp.full((LANES,), LANES - 1, jnp.int32)

        def dot16(t):
            prod = rows_v[slot, t, pl.ds(0, LANES)] * hv[0]
            for c in range(1, H // LANES):
                prod = prod + rows_v[slot, t, pl.ds(c * LANES, LANES)] * hv[c]
            csum = plsc.cumsum(prod)
            return _bcast_lane(csum, pick15)

        def blk(t0, n_tok):
            res = jnp.zeros((LANES,), jnp.float32)
            for k in range(n_tok):
                res = jnp.where(lane_iota == k, dot16(t0 + k), res)
            outb_v[slot, pl.ds(t0, LANES)] = res

        def blk_body(i, carry2):
            blk(i * LANES, LANES)
            return carry2

        lax.fori_loop(0, L // LANES, blk_body, 0, unroll=1)
        blk((L // LANES) * LANES, L - (L // LANES) * LANES)

        @pl.when(bl + NBUF < BPW)
        def _():
            fire_gather(bl + NBUF, slot)

        pltpu.async_copy(outb_v.at[slot, pl.ds(0, L)], out_hbm.at[b0 + bl], osem)
        return carry

    lax.fori_loop(0, BPW, batch_body, 0, unroll=1)

    for p in range(NBUF):
        bl = BPW - NBUF + p
        pltpu.make_async_copy(
            outb_v.at[lax.rem(jnp.int32(bl), NBUF), pl.ds(0, L)],
            out_hbm.at[b0 + bl],
            osem,
        ).wait()


def kernel(hidden_state, kb_arr, global_pointer, emb_table):
    del global_pointer
    kb = kb_arr.astype(jnp.int32)
    rep = _tc_repack(emb_table.T)
    rep_rows = rep.reshape(VOCP, H)
    return _entity_head(hidden_state, kb, rep_rows)


# speedup vs baseline: 6.7454x; 6.7454x over previous
import functools

import jax
import jax.numpy as jnp
from jax import lax
from jax.experimental import pallas as pl
from jax.experimental.pallas import tpu as pltpu
from jax.experimental.pallas import tpu_sc as plsc

B = 4096
L = 200
H = 64
VOC = 1000000
NC = 2
NS = 16
NW = NC * NS
BPW = B // NW
LANES = 16
NBUF = 4
LEG = 128

VBLK = 128
NBLK_MAIN = 244
V_MAIN = NBLK_MAIN * NW * VBLK

_PICK_DNUMS = lax.GatherDimensionNumbers(
    offset_dims=(), collapsed_slice_dims=(0,), start_index_map=(0,)
)


def _bcast_lane(vec, lane_idx):
    return lax.gather(
        vec,
        lane_idx[:, None],
        _PICK_DNUMS,
        (1,),
        mode=lax.GatherScatterMode.PROMISE_IN_BOUNDS,
    )


def _mesh():
    return plsc.VectorSubcoreMesh(
        core_axis_name="c", subcore_axis_name="s", num_cores=NC, num_subcores=NS
    )


NBLK = (VOC + 2 * H - 1) // (2 * H)
VOCP = NBLK * 2 * H


GRP = 13


def _tc_repack_body(x_ref, o_ref):
    x = x_ref[...]
    eye = jnp.eye(H, dtype=jnp.float32)
    xt = jax.lax.dot_general(
        x, eye, (((0,), (0,)), ((), ())), preferred_element_type=jnp.float32
    )
    for g in range(GRP):
        o_ref[pl.ds(H * g, H), 0:H] = xt[2 * H * g : 2 * H * g + H, :]
        o_ref[pl.ds(H * g, H), H : 2 * H] = xt[2 * H * g + H : 2 * H * (g + 1), :]


_tc_repack = pl.pallas_call(
    _tc_repack_body,
    grid=(NBLK // GRP,),
    in_specs=[pl.BlockSpec((H, GRP * 2 * H), lambda i: (0, i))],
    out_specs=pl.BlockSpec((GRP * H, 2 * H), lambda i: (i, 0)),
    out_shape=jax.ShapeDtypeStruct((VOCP // 2, 2 * H), jnp.float32),
)


@functools.partial(
    pl.kernel,
    out_type=jax.ShapeDtypeStruct((B, L), jnp.float32),
    mesh=_mesh(),
    compiler_params=pltpu.CompilerParams(
        needs_layout_passes=False, use_tc_tiling_on_sc=False
    ),
    scratch_types=[
        pltpu.VMEM((BPW, H), jnp.float32),
        pltpu.VMEM((BPW, L), jnp.int32),
        pltpu.VMEM((NBUF, 208), jnp.int32),
        pltpu.VMEM((NBUF, L, H), jnp.float32),
        pltpu.VMEM((NBUF, 208), jnp.float32),
        pltpu.SemaphoreType.DMA,
        pltpu.SemaphoreType.DMA,
    ],
)
def _entity_head(
    hid_hbm, kb_hbm, tab_hbm, out_hbm, hid_v, idx_v, idxs_v, rows_v, outb_v,
    gsem, osem,
):
    wid = lax.axis_index("s") * NC + lax.axis_index("c")
    b0 = wid * BPW
    pltpu.sync_copy(hid_hbm.at[pl.ds(b0, BPW)], hid_v)
    pltpu.sync_copy(kb_hbm.at[pl.ds(b0, BPW)], idx_v)

    def fire_gather(bl, slot):
        for k in range(L // LANES + 1):
            t0 = min(k * LANES, L - LANES)
            v = idx_v[bl, pl.ds(t0, LANES)]
            u = jnp.bitwise_and(v, 127)
            f = v + u - jnp.where(u >= H, 127, 0)
            idxs_v[slot, pl.ds(t0, LANES)] = f
        pltpu.async_copy(
            tab_hbm.at[idxs_v.at[slot, pl.ds(0, LEG)]],
            rows_v.at[slot, pl.ds(0, LEG)],
            gsem,
        )
        pltpu.async_copy(
            tab_hbm.at[idxs_v.at[slot, pl.ds(LEG, L - LEG)]],
            rows_v.at[slot, pl.ds(LEG, L - LEG)],
            gsem,
        )

    for p in range(NBUF):
        fire_gather(p, p)

    def batch_body(bl, carry):
        slot = lax.rem(bl, NBUF)
        pltpu.make_async_copy(
            tab_hbm.at[idxs_v.at[slot, pl.ds(0, LEG)]],
            rows_v.at[slot, pl.ds(0, LEG)],
            gsem,
        ).wait()
        pltpu.make_async_copy(
            tab_hbm.at[idxs_v.at[slot, pl.ds(LEG, L - LEG)]],
            rows_v.at[slot, pl.ds(LEG, L - LEG)],
            gsem,
        ).wait()

        @pl.when(bl >= NBUF)
        def _():
            pltpu.make_async_copy(
                outb_v.at[slot, pl.ds(0, L)], out_hbm.at[b0 + bl - NBUF], osem
            ).wait()

        hv = [hid_v[bl, pl.ds(c * LANES, LANES)] for c in range(H // LANES)]
        lane_iota = lax.iota(jnp.int32, LANES)
        pick15 = jnp.full((LANES,), LANES - 1, jnp.int32)

        def dot16(t):
            prod = rows_v[slot, t, pl.ds(0, LANES)] * hv[0]
            for c in range(1, H // LANES):
                prod = prod + rows_v[slot, t, pl.ds(c * LANES, LANES)] * hv[c]
            csum = plsc.cumsum(prod)
            return _bcast_lane(csum, pick15)

        def blk(t0, n_tok):
            res = jnp.zeros((LANES,), jnp.float32)
            for k in range(n_tok):
                res = jnp.where(lane_iota == k, dot16(t0 + k), res)
            outb_v[slot, pl.ds(t0, LANES)] = res

        def blk_body(i, carry2):
            blk(i * LANES, LANES)
            return carry2

        lax.fori_loop(0, L // LANES, blk_body, 0, unroll=1)
        blk((L // LANES) * LANES, L - (L // LANES) * LANES)

        @pl.when(bl + NBUF < BPW)
        def _():
            fire_gather(bl + NBUF, slot)

        pltpu.async_copy(outb_v.at[slot, pl.ds(0, L)], out_hbm.at[b0 + bl], osem)
        return carry

    lax.fori_loop(0, BPW, batch_body, 0, unroll=1)

    for p in range(NBUF):
        bl = BPW - NBUF + p
        pltpu.make_async_copy(
            outb_v.at[lax.rem(jnp.int32(bl), NBUF), pl.ds(0, L)],
            out_hbm.at[b0 + bl],
            osem,
        ).wait()


def kernel(hidden_state, kb_arr, global_pointer, emb_table):
    del global_pointer
    kb = kb_arr.astype(jnp.int32)
    rep = _tc_repack(emb_table.T)
    rep_rows = rep.reshape(VOCP, H)
    return _entity_head(hidden_state, kb, rep_rows)
